# Initial kernel scaffold; baseline (speedup 1.0000x reference)
import functools

import jax
import jax.numpy as jnp
from jax import lax
from jax.experimental import pallas as pl
from jax.experimental.pallas import tpu as pltpu
from jax.experimental.pallas import tpu_sc as plsc

XD = 256
N = 512 * 512
Q = 32
NC, NS, L = 2, 16, 16
NW = NC * NS
PPW = N // NW
C = 128
NCHUNK = PPW // C


def _body(m_ref, x0_ref, y0_ref, x1_ref, y1_ref, wx_ref, wy_ref, out_ref,
          x0_v, y0_v, x1_v, y1_v, wx_v, wy_v,
          w00_v, w01_v, w10_v, w11_v,
          i00_v, i01_v, i10_v, i11_v,
          r00_v, r01_v, r10_v, r11_v, out_v,
          s0, s1, s2, s3):
    wid = lax.axis_index("s") * NC + lax.axis_index("c")
    base = wid * PPW

    def chunk(g, carry):
        off = base + g * C
        sl = pl.ds(off, C)
        pltpu.sync_copy(x0_ref.at[sl], x0_v)
        pltpu.sync_copy(y0_ref.at[sl], y0_v)
        pltpu.sync_copy(x1_ref.at[sl], x1_v)
        pltpu.sync_copy(y1_ref.at[sl], y1_v)
        pltpu.sync_copy(wx_ref.at[sl], wx_v)
        pltpu.sync_copy(wy_ref.at[sl], wy_v)

        def prep(j, carry2):
            s = pl.ds(j * L, L)
            xx0 = x0_v[s]
            yy0 = y0_v[s]
            xx1 = x1_v[s]
            yy1 = y1_v[s]
            i00_v[s] = yy0 * XD + xx0
            i01_v[s] = yy0 * XD + xx1
            i10_v[s] = yy1 * XD + xx0
            i11_v[s] = yy1 * XD + xx1
            ax = wx_v[s]
            ay = wy_v[s]
            bx = 1.0 - ax
            by = 1.0 - ay
            w00_v[s] = bx * by
            w01_v[s] = ax * by
            w10_v[s] = bx * ay
            w11_v[s] = ax * ay
            return carry2

        lax.fori_loop(0, C // L, prep, 0)

        c0 = pltpu.async_copy(m_ref.at[i00_v], r00_v, s0)
        c1 = pltpu.async_copy(m_ref.at[i01_v], r01_v, s1)
        c2 = pltpu.async_copy(m_ref.at[i10_v], r10_v, s2)
        c3 = pltpu.async_copy(m_ref.at[i11_v], r11_v, s3)
        c0.wait()
        c1.wait()
        c2.wait()
        c3.wait()

        lo = pl.ds(0, L)
        hi = pl.ds(L, L)

        def point(p, carry2):
            a = w00_v[p]
            b = w01_v[p]
            c = w10_v[p]
            d = w11_v[p]
            out_v[p, lo] = (a * r00_v[p, lo] + b * r01_v[p, lo]
                            + c * r10_v[p, lo] + d * r11_v[p, lo])
            out_v[p, hi] = (a * r00_v[p, hi] + b * r01_v[p, hi]
                            + c * r10_v[p, hi] + d * r11_v[p, hi])
            return carry2

        lax.fori_loop(0, C, point, 0)

        pltpu.sync_copy(out_v, out_ref.at[sl])
        return carry

    lax.fori_loop(0, NCHUNK, chunk, 0)


@jax.jit
def _run(m3, x0, y0, x1, y1, wx, wy):
    mesh = plsc.VectorSubcoreMesh(
        core_axis_name="c", subcore_axis_name="s",
        num_cores=NC, num_subcores=NS)
    f = pl.kernel(
        _body,
        out_type=jax.ShapeDtypeStruct((N, Q), jnp.float32),
        mesh=mesh,
        scratch_types=[
            pltpu.VMEM((C,), jnp.int32),
            pltpu.VMEM((C,), jnp.int32),
            pltpu.VMEM((C,), jnp.int32),
            pltpu.VMEM((C,), jnp.int32),
            pltpu.VMEM((C,), jnp.float32),
            pltpu.VMEM((C,), jnp.float32),
            pltpu.VMEM((C,), jnp.float32),
            pltpu.VMEM((C,), jnp.float32),
            pltpu.VMEM((C,), jnp.float32),
            pltpu.VMEM((C,), jnp.float32),
            pltpu.VMEM((C,), jnp.int32),
            pltpu.VMEM((C,), jnp.int32),
            pltpu.VMEM((C,), jnp.int32),
            pltpu.VMEM((C,), jnp.int32),
            pltpu.VMEM((C, Q), jnp.float32),
            pltpu.VMEM((C, Q), jnp.float32),
            pltpu.VMEM((C, Q), jnp.float32),
            pltpu.VMEM((C, Q), jnp.float32),
            pltpu.VMEM((C, Q), jnp.float32),
            pltpu.SemaphoreType.DMA,
            pltpu.SemaphoreType.DMA,
            pltpu.SemaphoreType.DMA,
            pltpu.SemaphoreType.DMA,
        ],
    )
    return f(m3, x0, y0, x1, y1, wx, wy)


def kernel(M, x0, y0, x1, y1, wx, wy):
    m3 = M.reshape(-1, Q)
    return _run(m3, x0, y0, x1, y1, wx.reshape(-1), wy.reshape(-1))


# speedup vs baseline: 35.9477x; 35.9477x over previous
import functools

import jax
import jax.numpy as jnp
from jax import lax
from jax.experimental import pallas as pl
from jax.experimental.pallas import tpu as pltpu
from jax.experimental.pallas import tpu_sc as plsc

XD = 256
N = 512 * 512
Q = 32
NC, NS, L = 2, 16, 16
NW = NC * NS
PPW = N // NW
C = 128
NCHUNK = PPW // C


def _body(m_ref, x0_ref, y0_ref, x1_ref, y1_ref, wx_ref, wy_ref, out_ref,
          x0_v, y0_v, x1_v, y1_v, wx_v, wy_v,
          w00_v, w01_v, w10_v, w11_v,
          i00_v, i01_v, i10_v, i11_v,
          r00_v, r01_v, r10_v, r11_v, out_v,
          s0, s1, s2, s3):
    wid = lax.axis_index("s") * NC + lax.axis_index("c")
    base = wid * PPW

    def chunk(g, carry):
        off = base + g * C
        sl = pl.ds(off, C)
        pltpu.sync_copy(x0_ref.at[sl], x0_v)
        pltpu.sync_copy(y0_ref.at[sl], y0_v)
        pltpu.sync_copy(x1_ref.at[sl], x1_v)
        pltpu.sync_copy(y1_ref.at[sl], y1_v)
        pltpu.sync_copy(wx_ref.at[sl], wx_v)
        pltpu.sync_copy(wy_ref.at[sl], wy_v)

        def prep(j, carry2):
            s = pl.ds(j * L, L)
            xx0 = x0_v[s]
            yy0 = y0_v[s]
            xx1 = x1_v[s]
            yy1 = y1_v[s]
            i00_v[s] = yy0 * XD + xx0
            i01_v[s] = yy0 * XD + xx1
            i10_v[s] = yy1 * XD + xx0
            i11_v[s] = yy1 * XD + xx1
            ax = wx_v[s]
            ay = wy_v[s]
            bx = 1.0 - ax
            by = 1.0 - ay
            w00_v[s] = bx * by
            w01_v[s] = ax * by
            w10_v[s] = bx * ay
            w11_v[s] = ax * ay
            return carry2

        lax.fori_loop(0, C // L, prep, 0)

        c0 = pltpu.async_copy(m_ref.at[i00_v], r00_v, s0)
        c1 = pltpu.async_copy(m_ref.at[i01_v], r01_v, s1)
        c2 = pltpu.async_copy(m_ref.at[i10_v], r10_v, s2)
        c3 = pltpu.async_copy(m_ref.at[i11_v], r11_v, s3)
        c0.wait()
        c1.wait()
        c2.wait()
        c3.wait()

        lo = pl.ds(0, L)
        hi = pl.ds(L, L)

        def point16(j, carry2):
            p0 = j * L
            wa = w00_v[pl.ds(p0, L)]
            wb = w01_v[pl.ds(p0, L)]
            wc = w10_v[pl.ds(p0, L)]
            wd = w11_v[pl.ds(p0, L)]
            for k in range(L):
                p = p0 + k
                a = wa[k]
                b = wb[k]
                c = wc[k]
                d = wd[k]
                out_v[p, lo] = (a * r00_v[p, lo] + b * r01_v[p, lo]
                                + c * r10_v[p, lo] + d * r11_v[p, lo])
                out_v[p, hi] = (a * r00_v[p, hi] + b * r01_v[p, hi]
                                + c * r10_v[p, hi] + d * r11_v[p, hi])
            return carry2

        lax.fori_loop(0, C // L, point16, 0)

        pltpu.sync_copy(out_v, out_ref.at[sl])
        return carry

    lax.fori_loop(0, NCHUNK, chunk, 0)


@jax.jit
def _run(m3, x0, y0, x1, y1, wx, wy):
    mesh = plsc.VectorSubcoreMesh(
        core_axis_name="c", subcore_axis_name="s",
        num_cores=NC, num_subcores=NS)
    f = pl.kernel(
        _body,
        out_type=jax.ShapeDtypeStruct((N, Q), jnp.float32),
        mesh=mesh,
        compiler_params=pltpu.CompilerParams(use_tc_tiling_on_sc=False),
        scratch_types=[
            pltpu.VMEM((C,), jnp.int32),
            pltpu.VMEM((C,), jnp.int32),
            pltpu.VMEM((C,), jnp.int32),
            pltpu.VMEM((C,), jnp.int32),
            pltpu.VMEM((C,), jnp.float32),
            pltpu.VMEM((C,), jnp.float32),
            pltpu.VMEM((C,), jnp.float32),
            pltpu.VMEM((C,), jnp.float32),
            pltpu.VMEM((C,), jnp.float32),
            pltpu.VMEM((C,), jnp.float32),
            pltpu.VMEM((C,), jnp.int32),
            pltpu.VMEM((C,), jnp.int32),
            pltpu.VMEM((C,), jnp.int32),
            pltpu.VMEM((C,), jnp.int32),
            pltpu.VMEM((C, Q), jnp.float32),
            pltpu.VMEM((C, Q), jnp.float32),
            pltpu.VMEM((C, Q), jnp.float32),
            pltpu.VMEM((C, Q), jnp.float32),
            pltpu.VMEM((C, Q), jnp.float32),
            pltpu.SemaphoreType.DMA,
            pltpu.SemaphoreType.DMA,
            pltpu.SemaphoreType.DMA,
            pltpu.SemaphoreType.DMA,
        ],
    )
    return f(m3, x0, y0, x1, y1, wx, wy)


def kernel(M, x0, y0, x1, y1, wx, wy):
    m3 = M.reshape(-1, Q)
    return _run(m3, x0, y0, x1, y1, wx.reshape(-1), wy.reshape(-1))


# speedup vs baseline: 68.1892x; 1.8969x over previous
import jax
import jax.numpy as jnp
from jax import lax
from jax.experimental import pallas as pl
from jax.experimental.pallas import tpu as pltpu
from jax.experimental.pallas import tpu_sc as plsc

XD = 256
N = 512 * 512
Q = 32
NC, NS, L = 2, 16, 16
NW = NC * NS
PPW = N // NW
C = 128
NCHUNK = PPW // C


def _body(m_ref, x0_ref, y0_ref, x1_ref, y1_ref, wx_ref, wy_ref, out_ref,
          x0_v, y0_v, x1_v, y1_v, wx_v, wy_v, we0_v, we1_v,
          r00_v, r01_v, r10_v, r11_v, out_v,
          si, s0, s1, o0, o1):
    wid = lax.axis_index("s") * NC + lax.axis_index("c")
    base = wid * PPW
    wsl = pl.ds(base, PPW)

    stages = [
        pltpu.async_copy(x0_ref.at[wsl], x0_v, si),
        pltpu.async_copy(y0_ref.at[wsl], y0_v, si),
        pltpu.async_copy(x1_ref.at[wsl], x1_v, si),
        pltpu.async_copy(y1_ref.at[wsl], y1_v, si),
        pltpu.async_copy(wx_ref.at[wsl], wx_v, si),
        pltpu.async_copy(wy_ref.at[wsl], wy_v, si),
    ]
    for c in stages:
        c.wait()

    def prep(j, carry):
        s = pl.ds(j * L, L)
        xx0 = x0_v[s]
        yy0 = y0_v[s]
        xx1 = x1_v[s]
        yy1 = y1_v[s]
        ax = wx_v[s]
        ay = wy_v[s]
        x0_v[s] = yy0 * XD + xx0
        x1_v[s] = yy0 * XD + xx1
        y0_v[s] = yy1 * XD + xx0
        y1_v[s] = yy1 * XD + xx1
        bx = 1.0 - ax
        by = 1.0 - ay
        wx_v[s] = bx * by
        wy_v[s] = ax * by
        we0_v[s] = bx * ay
        we1_v[s] = ax * ay
        return carry

    lax.fori_loop(0, PPW // L, prep, 0)

    sems = (s0, s1)
    osems = (o0, o1)
    rows = (r00_v, r01_v, r10_v, r11_v)
    idxs = (x0_v, x1_v, y0_v, y1_v)
    wgts = (wx_v, wy_v, we0_v, we1_v)

    def fire(g, b):
        off = g * C
        for t in range(4):
            pltpu.async_copy(m_ref.at[idxs[t].at[pl.ds(off, C)]],
                             rows[t].at[b], sems[b])

    def drain(b):
        for t in range(4):
            pltpu.make_async_copy(m_ref.at[pl.ds(0, C)],
                                  rows[t].at[b], sems[b]).wait()

    def out_desc(g, b):
        return pltpu.make_async_copy(
            out_v.at[b], out_ref.at[pl.ds(base + g * C, C)], osems[b])

    fire(0, 0)
    lo = pl.ds(0, L)
    hi = pl.ds(L, L)

    def iter_body(i, carry):
        for b in range(2):
            g = 2 * i + b

            @pl.when(g + 1 < NCHUNK)
            def _():
                fire(g + 1, 1 - b)

            drain(b)

            @pl.when(g >= 2)
            def _():
                out_desc(g, b).wait()

            def point16(j, carry2):
                q0 = g * C + j * L
                p0 = j * L
                wa = wgts[0][pl.ds(q0, L)]
                wb = wgts[1][pl.ds(q0, L)]
                wc = wgts[2][pl.ds(q0, L)]
                wd = wgts[3][pl.ds(q0, L)]
                for k in range(L):
                    p = p0 + k
                    a = wa[k]
                    bb = wb[k]
                    cc = wc[k]
                    dd = wd[k]
                    out_v[b, p, lo] = (a * r00_v[b, p, lo]
                                       + bb * r01_v[b, p, lo]
                                       + cc * r10_v[b, p, lo]
                                       + dd * r11_v[b, p, lo])
                    out_v[b, p, hi] = (a * r00_v[b, p, hi]
                                       + bb * r01_v[b, p, hi]
                                       + cc * r10_v[b, p, hi]
                                       + dd * r11_v[b, p, hi])
                return carry2

            lax.fori_loop(0, C // L, point16, 0)
            pltpu.async_copy(out_v.at[b],
                             out_ref.at[pl.ds(base + g * C, C)], osems[b])
        return carry

    lax.fori_loop(0, NCHUNK // 2, iter_body, 0)
    out_desc(NCHUNK - 2, 0).wait()
    out_desc(NCHUNK - 1, 1).wait()


@jax.jit
def _run(m3, x0, y0, x1, y1, wx, wy):
    mesh = plsc.VectorSubcoreMesh(
        core_axis_name="c", subcore_axis_name="s",
        num_cores=NC, num_subcores=NS)
    f = pl.kernel(
        _body,
        out_type=jax.ShapeDtypeStruct((N, Q), jnp.float32),
        mesh=mesh,
        compiler_params=pltpu.CompilerParams(use_tc_tiling_on_sc=False),
        scratch_types=[
            pltpu.VMEM((PPW,), jnp.int32),
            pltpu.VMEM((PPW,), jnp.int32),
            pltpu.VMEM((PPW,), jnp.int32),
            pltpu.VMEM((PPW,), jnp.int32),
            pltpu.VMEM((PPW,), jnp.float32),
            pltpu.VMEM((PPW,), jnp.float32),
            pltpu.VMEM((PPW,), jnp.float32),
            pltpu.VMEM((PPW,), jnp.float32),
            pltpu.VMEM((2, C, Q), jnp.float32),
            pltpu.VMEM((2, C, Q), jnp.float32),
            pltpu.VMEM((2, C, Q), jnp.float32),
            pltpu.VMEM((2, C, Q), jnp.float32),
            pltpu.VMEM((2, C, Q), jnp.float32),
            pltpu.SemaphoreType.DMA,
            pltpu.SemaphoreType.DMA,
            pltpu.SemaphoreType.DMA,
            pltpu.SemaphoreType.DMA,
            pltpu.SemaphoreType.DMA,
        ],
    )
    return f(m3, x0, y0, x1, y1, wx, wy)


def kernel(M, x0, y0, x1, y1, wx, wy):
    m3 = M.reshape(-1, Q)
    return _run(m3, x0, y0, x1, y1, wx.reshape(-1), wy.reshape(-1))
